# Initial kernel scaffold; baseline (speedup 1.0000x reference)
import functools

import jax
import jax.numpy as jnp
from jax.experimental import pallas as pl
from jax.experimental.pallas import tpu as pltpu

ZD, YD, XD = 41, 80, 80
EPS = 1e-3
F32 = jnp.float32


def _padv(a, pads):
    for ax, (lo, hi) in enumerate(pads):
        if lo == 0 and hi == 0:
            continue
        parts = []
        if lo:
            sh = list(a.shape)
            sh[ax] = lo
            parts.append(jnp.zeros(sh, a.dtype))
        parts.append(a)
        if hi:
            sh = list(a.shape)
            sh[ax] = hi
            parts.append(jnp.zeros(sh, a.dtype))
        a = jnp.concatenate(parts, axis=ax)
    return a


def _conv_vol(a, w, ks, st, oshape):
    kz, ky, kx = ks
    sz, sy, sx = st
    Zo, Yo, Xo = oshape
    Ci = a.shape[-1]
    Co = w.shape[-1]
    acc = jnp.zeros((Zo * Yo * Xo, Co), F32)
    for dz in range(kz):
        for dy in range(ky):
            for dx in range(kx):
                sl = jax.lax.slice(
                    a, (dz, dy, dx, 0),
                    (dz + sz * (Zo - 1) + 1, dy + sy * (Yo - 1) + 1,
                     dx + sx * (Xo - 1) + 1, Ci),
                    (sz, sy, sx, 1))
                acc = acc + jnp.dot(sl.reshape(Zo * Yo * Xo, Ci),
                                    w[dz * ky * kx + dy * kx + dx],
                                    preferred_element_type=F32)
    return acc.reshape(Zo, Yo, Xo, Co)


def _mask_vol(m, ks, st, oshape):
    kz, ky, kx = ks
    sz, sy, sx = st
    Zo, Yo, Xo = oshape
    mc = jnp.zeros((Zo, Yo, Xo), F32)
    for dz in range(kz):
        for dy in range(ky):
            for dx in range(kx):
                mc = mc + jax.lax.slice(
                    m, (dz, dy, dx),
                    (dz + sz * (Zo - 1) + 1, dy + sy * (Yo - 1) + 1,
                     dx + sx * (Xo - 1) + 1),
                    (sz, sy, sx))
    return (mc > 0.5).astype(F32)


def _pack_partials(y, nm):
    Co = y.shape[-1]
    mm = nm[..., None]
    red = tuple(range(y.ndim - 1))
    s1 = jnp.sum(y * mm, red)
    s2 = jnp.sum(y * y * mm, red)
    cm = jnp.sum(nm)
    pad = jnp.zeros((128 - Co,), F32)
    r0 = jnp.concatenate([s1, pad])[None]
    r1 = jnp.concatenate([s2, pad])[None]
    r2 = jnp.full((1, 128), cm, F32)
    return jnp.concatenate([r0, r1, r2, jnp.zeros((5, 128), F32)], axis=0)


def _finalize(part, g, b):
    Co = g.shape[0]
    cnt = jnp.maximum(part[2, 0], 1.0)
    mean = part[0, :Co] / cnt
    var = jnp.maximum(part[1, :Co] / cnt - mean * mean, 0.0)
    scale = g * jax.lax.rsqrt(var + EPS)
    shift = b - mean * scale
    return scale[None], shift[None]


def _bn_act(y, nm, g, b):
    mm = nm[..., None]
    cnt = jnp.maximum(jnp.sum(nm), 1.0)
    mean = jnp.sum(y * mm, (0, 1, 2)) / cnt
    var = jnp.maximum(jnp.sum(y * y * mm, (0, 1, 2)) / cnt - mean * mean, 0.0)
    scale = g * jax.lax.rsqrt(var + EPS)
    shift = b - mean * scale
    return jnp.maximum(y * scale + shift, 0.0) * mm


def _first_subm(xp, mp, w):
    Zp, Yp, Xp, Ci = xp.shape
    Co = w.shape[-1]
    Zo, Yo, Xo = Zp - 2, Yp - 2, Xp - 2

    def body(x0, x1, x2, m1, wr, yref, pref):
        zi = pl.program_id(0)
        acc = jnp.zeros((Yo * Xo, Co), F32)
        for dz, xr in enumerate((x0, x1, x2)):
            a = xr[0]
            for dy in range(3):
                for dx in range(3):
                    sl = jax.lax.slice(a, (dy, dx, 0), (dy + Yo, dx + Xo, Ci))
                    acc = acc + jnp.dot(sl.reshape(Yo * Xo, Ci),
                                        wr[dz * 9 + dy * 3 + dx],
                                        preferred_element_type=F32)
        mc = jax.lax.slice(m1[0], (1, 1), (1 + Yo, 1 + Xo))
        accv = acc.reshape(Yo, Xo, Co)
        yref[0] = accv
        blk = _pack_partials(accv, mc)

        @pl.when(zi == 0)
        def _():
            pref[...] = blk

        @pl.when(zi != 0)
        def _():
            pref[...] = pref[...] + blk

    return pl.pallas_call(
        body,
        grid=(Zo,),
        in_specs=[
            pl.BlockSpec((1, Yp, Xp, Ci), lambda z: (z, 0, 0, 0)),
            pl.BlockSpec((1, Yp, Xp, Ci), lambda z: (z + 1, 0, 0, 0)),
            pl.BlockSpec((1, Yp, Xp, Ci), lambda z: (z + 2, 0, 0, 0)),
            pl.BlockSpec((1, Yp, Xp), lambda z: (z + 1, 0, 0)),
            pl.BlockSpec((27, Ci, Co), lambda z: (0, 0, 0)),
        ],
        out_specs=[
            pl.BlockSpec((1, Yo, Xo, Co), lambda z: (z, 0, 0, 0)),
            pl.BlockSpec((8, 128), lambda z: (0, 0)),
        ],
        out_shape=[
            jax.ShapeDtypeStruct((Zo, Yo, Xo, Co), F32),
            jax.ShapeDtypeStruct((8, 128), F32),
        ],
    )(xp, xp, xp, mp, w)


def _grid_spconv(xp, mp, sc, sh, w, stride, Zo):
    Zp, Yp, Xp, Ci = xp.shape
    Co = w.shape[-1]
    s = stride
    Yo = (Yp - 3) // s + 1
    Xo = (Xp - 3) // s + 1

    def body(x0, x1, x2, m0, m1, m2, scr, shr, wr, yref, nmref, pref):
        zi = pl.program_id(0)
        scv = scr[0]
        shv = shr[0]
        acc = jnp.zeros((Yo * Xo, Co), F32)
        mc = jnp.zeros((Yo, Xo), F32)
        for dz, (xr, mr) in enumerate(((x0, m0), (x1, m1), (x2, m2))):
            mv = mr[0]
            a = jnp.maximum(xr[0] * scv + shv, 0.0) * mv[..., None]
            for dy in range(3):
                for dx in range(3):
                    sl = jax.lax.slice(
                        a, (dy, dx, 0),
                        (dy + s * (Yo - 1) + 1, dx + s * (Xo - 1) + 1, Ci),
                        (s, s, 1))
                    acc = acc + jnp.dot(sl.reshape(Yo * Xo, Ci),
                                        wr[dz * 9 + dy * 3 + dx],
                                        preferred_element_type=F32)
                    mc = mc + jax.lax.slice(
                        mv, (dy, dx),
                        (dy + s * (Yo - 1) + 1, dx + s * (Xo - 1) + 1), (s, s))
        nm = (mc > 0.5).astype(F32)
        accv = acc.reshape(Yo, Xo, Co)
        yref[0] = accv
        nmref[0] = nm
        blk = _pack_partials(accv, nm)

        @pl.when(zi == 0)
        def _():
            pref[...] = blk

        @pl.when(zi != 0)
        def _():
            pref[...] = pref[...] + blk

    return pl.pallas_call(
        body,
        grid=(Zo,),
        in_specs=[
            pl.BlockSpec((1, Yp, Xp, Ci), lambda z: (s * z, 0, 0, 0)),
            pl.BlockSpec((1, Yp, Xp, Ci), lambda z: (s * z + 1, 0, 0, 0)),
            pl.BlockSpec((1, Yp, Xp, Ci), lambda z: (s * z + 2, 0, 0, 0)),
            pl.BlockSpec((1, Yp, Xp), lambda z: (s * z, 0, 0)),
            pl.BlockSpec((1, Yp, Xp), lambda z: (s * z + 1, 0, 0)),
            pl.BlockSpec((1, Yp, Xp), lambda z: (s * z + 2, 0, 0)),
            pl.BlockSpec((1, Ci), lambda z: (0, 0)),
            pl.BlockSpec((1, Ci), lambda z: (0, 0)),
            pl.BlockSpec((27, Ci, Co), lambda z: (0, 0, 0)),
        ],
        out_specs=[
            pl.BlockSpec((1, Yo, Xo, Co), lambda z: (z, 0, 0, 0)),
            pl.BlockSpec((1, Yo, Xo), lambda z: (z, 0, 0)),
            pl.BlockSpec((8, 128), lambda z: (0, 0)),
        ],
        out_shape=[
            jax.ShapeDtypeStruct((Zo, Yo, Xo, Co), F32),
            jax.ShapeDtypeStruct((Zo, Yo, Xo), F32),
            jax.ShapeDtypeStruct((8, 128), F32),
        ],
    )(xp, xp, xp, mp, mp, mp, sc, sh, w)


def _single_conv(xp, mp, sc, sh, w, stride, subm):
    Zp, Yp, Xp, Ci = xp.shape
    Co = w.shape[-1]
    sz, sy, sx = stride
    Zo = (Zp - 3) // sz + 1
    Yo = (Yp - 3) // sy + 1
    Xo = (Xp - 3) // sx + 1

    def body(xr, mr, scr, shr, wr, yref, nmref, pref):
        mv = mr[...]
        a = jnp.maximum(xr[...] * scr[0] + shr[0], 0.0) * mv[..., None]
        y = _conv_vol(a, wr[...], (3, 3, 3), stride, (Zo, Yo, Xo))
        if subm:
            nm = jax.lax.slice(mv, (1, 1, 1), (1 + Zo, 1 + Yo, 1 + Xo))
        else:
            nm = _mask_vol(mv, (3, 3, 3), stride, (Zo, Yo, Xo))
        yref[...] = y
        nmref[...] = nm
        pref[...] = _pack_partials(y, nm)

    return pl.pallas_call(
        body,
        out_shape=[
            jax.ShapeDtypeStruct((Zo, Yo, Xo, Co), F32),
            jax.ShapeDtypeStruct((Zo, Yo, Xo), F32),
            jax.ShapeDtypeStruct((8, 128), F32),
        ],
    )(xp, mp, sc, sh, w)


def _tail(y2c, nm2c, sc, sh, ws, gs, bs):
    def body(yr, mr, scr, shr, w3a, w3b, w3c, w4a, w4b, w4c, wo,
             g3a, g3b, g3c, g4a, g4b, g4c, go,
             b3a, b3b, b3c, b4a, b4b, b4c, bo, outref):
        m2 = mr[...]
        a = jnp.maximum(yr[...] * scr[0] + shr[0], 0.0) * m2[..., None]
        ap = _padv(a, ((1, 1), (1, 1), (1, 1), (0, 0)))
        mp = _padv(m2, ((1, 1), (1, 1), (1, 1)))
        y = _conv_vol(ap, w3a[...], (3, 3, 3), (2, 2, 2), (11, 20, 20))
        nm3 = _mask_vol(mp, (3, 3, 3), (2, 2, 2), (11, 20, 20))
        a = _bn_act(y, nm3, g3a[0], b3a[0])
        for wr, gr, br in ((w3b, g3b, b3b), (w3c, g3c, b3c)):
            ap = _padv(a, ((1, 1), (1, 1), (1, 1), (0, 0)))
            y = _conv_vol(ap, wr[...], (3, 3, 3), (1, 1, 1), (11, 20, 20))
            a = _bn_act(y, nm3, gr[0], br[0])
        ap = _padv(a, ((0, 0), (1, 1), (1, 1), (0, 0)))
        mp4 = _padv(nm3, ((0, 0), (1, 1), (1, 1)))
        y = _conv_vol(ap, w4a[...], (3, 3, 3), (2, 2, 2), (5, 10, 10))
        nm4 = _mask_vol(mp4, (3, 3, 3), (2, 2, 2), (5, 10, 10))
        a = _bn_act(y, nm4, g4a[0], b4a[0])
        for wr, gr, br in ((w4b, g4b, b4b), (w4c, g4c, b4c)):
            ap = _padv(a, ((1, 1), (1, 1), (1, 1), (0, 0)))
            y = _conv_vol(ap, wr[...], (3, 3, 3), (1, 1, 1), (5, 10, 10))
            a = _bn_act(y, nm4, gr[0], br[0])
        y = _conv_vol(a, wo[...], (3, 1, 1), (2, 1, 1), (2, 10, 10))
        nmo = _mask_vol(nm4, (3, 1, 1), (2, 1, 1), (2, 10, 10))
        outref[...] = _bn_act(y, nmo, go[0], bo[0])

    return pl.pallas_call(
        body,
        out_shape=jax.ShapeDtypeStruct((2, 10, 10, 128), F32),
    )(y2c, nm2c, sc, sh, *ws, *gs, *bs)


def kernel(voxel_features, voxel_coords, batch_size, params):
    p = params
    feats = voxel_features.astype(F32)
    zi = voxel_coords[:, 1]
    yi = voxel_coords[:, 2]
    xi = voxel_coords[:, 3]
    dense = jnp.zeros((ZD, YD, XD, 4), F32).at[zi, yi, xi].set(feats)
    mask = jnp.zeros((ZD, YD, XD), F32).at[zi, yi, xi].set(1.0)

    pad4 = lambda a: jnp.pad(a, ((1, 1), (1, 1), (1, 1), (0, 0)))
    pad3 = lambda a: jnp.pad(a, ((1, 1), (1, 1), (1, 1)))
    rw = lambda w: w.reshape(27, w.shape[3], w.shape[4])

    mp0 = pad3(mask)
    y, part = _first_subm(pad4(dense), mp0, rw(p['win']))
    sc, sh = _finalize(part, p['gin'], p['bin'])
    y, nm, part = _grid_spconv(pad4(y), mp0, sc, sh, rw(p['w1']), 1, ZD)
    sc, sh = _finalize(part, p['g1'], p['b1'])
    y, nm, part = _grid_spconv(pad4(y), pad3(nm), sc, sh, rw(p['w2a']), 2, 21)
    sc, sh = _finalize(part, p['g2a'], p['b2a'])
    mp = pad3(nm)
    y, nm, part = _single_conv(pad4(y), mp, sc, sh, rw(p['w2b']), (1, 1, 1),
                               subm=True)
    sc, sh = _finalize(part, p['g2b'], p['b2b'])
    y, nm, part = _single_conv(pad4(y), mp, sc, sh, rw(p['w2c']), (1, 1, 1),
                               subm=False)
    sc, sh = _finalize(part, p['g2c'], p['b2c'])
    ws = [rw(p['w3a']), rw(p['w3b']), rw(p['w3c']), rw(p['w4a']),
          rw(p['w4b']), rw(p['w4c']), p['wout'].reshape(3, 64, 128)]
    gs = [p[k][None] for k in ('g3a', 'g3b', 'g3c', 'g4a', 'g4b', 'g4c',
                               'gout')]
    bs = [p[k][None] for k in ('b3a', 'b3b', 'b3c', 'b4a', 'b4b', 'b4c',
                               'bout')]
    out = _tail(y, nm, sc, sh, ws, gs, bs)
    return out[None]


# speedup vs baseline: 1.7093x; 1.7093x over previous
import jax
import jax.numpy as jnp
from jax.experimental import pallas as pl

ZD, YD, XD = 41, 80, 80
EPS = 1e-3
F32 = jnp.float32


def _conv_layer(x, m, wT, *, Zi, Zo, s, zoff, Wf, kyx, act, subm,
                sc=None, sh=None, sel=None):
    G = Wf + 1
    P = Wf * Wf
    M = P + 2 * G
    _, Ci, _ = x.shape
    Co = wT.shape[0]
    taps = [(dy, dx) for dy in range(kyx) for dx in range(kyx)]

    def body(*refs):
        xr = refs[0:3]
        mr = refs[3:6]
        i = 6
        if act:
            scr, shr = refs[i], refs[i + 1]
            i += 2
        if sel is not None:
            selr = refs[i]
            i += 1
        wr = refs[i]
        yref, nmref, pref = refs[i + 1:i + 4]

        zi = pl.program_id(0)
        mc = jnp.zeros((1, P), F32)
        m_center = None
        sls = []
        for dz in range(3):
            vz = s * zi + zoff + dz
            valid = jnp.where((vz >= 0) & (vz < Zi), 1.0, 0.0)
            mv = mr[dz][0] * valid
            if dz == 1:
                m_center = mr[dz][0]
            xv = xr[dz][0]
            if act:
                av = jnp.maximum(xv * scr[...] + shr[...], 0.0) * mv
            else:
                av = xv * valid
            for dy, dx in taps:
                o = (dy - 1) * Wf + (dx - 1) if kyx == 3 else 0
                sls.append(jax.lax.slice(av, (0, G + o), (Ci, G + o + P)))
                if not subm:
                    mc = mc + jax.lax.slice(mv, (0, G + o), (1, G + o + P))
        rows = sls[0] if len(sls) == 1 else jnp.concatenate(sls, axis=0)
        acc = jnp.dot(wr[...], rows, preferred_element_type=F32)
        if subm:
            nm_core = jax.lax.slice(m_center, (0, G), (1, G + P))
            nmfull = m_center
        else:
            nm_core = (mc > 0.5).astype(F32)
            nm_core = nm_core * jax.lax.slice(selr[...], (0, G), (1, G + P))
            zg = jnp.zeros((1, G), F32)
            nmfull = jnp.concatenate([zg, nm_core, zg], axis=1)
        zgc = jnp.zeros((Co, G), F32)
        yref[0] = jnp.concatenate([zgc, acc, zgc], axis=1)
        nmref[0] = nmfull

        nm_eff = nm_core
        s1 = jnp.sum(acc * nm_eff, axis=1, keepdims=True)
        s2 = jnp.sum(acc * acc * nm_eff, axis=1, keepdims=True)
        cm = jnp.zeros((Co, 1), F32) + jnp.sum(nm_eff)
        blk = jnp.concatenate([s1, s2, cm, jnp.zeros((Co, 5), F32)], axis=1)
        if Co < 128:
            blk = jnp.concatenate([blk, jnp.zeros((128 - Co, 8), F32)],
                                  axis=0)

        @pl.when(zi == 0)
        def _():
            pref[...] = blk

        @pl.when(zi != 0)
        def _():
            pref[...] = pref[...] + blk

    def xmap(d):
        return lambda z: (jnp.clip(s * z + zoff + d, 0, Zi - 1), 0, 0)

    def mmap(d):
        return lambda z: (jnp.clip(s * z + zoff + d, 0, Zi - 1), 0, 0)

    in_specs = [pl.BlockSpec((1, Ci, M), xmap(d)) for d in range(3)]
    in_specs += [pl.BlockSpec((1, 1, M), mmap(d)) for d in range(3)]
    args = [x, x, x, m, m, m]
    if act:
        in_specs += [pl.BlockSpec((Ci, 1), lambda z: (0, 0))] * 2
        args += [sc, sh]
    if sel is not None:
        in_specs += [pl.BlockSpec((1, M), lambda z: (0, 0))]
        args += [sel]
    in_specs += [pl.BlockSpec(wT.shape, lambda z: (0, 0))]
    args += [wT]

    return pl.pallas_call(
        body,
        grid=(Zo,),
        in_specs=in_specs,
        out_specs=[
            pl.BlockSpec((1, Co, M), lambda z: (z, 0, 0)),
            pl.BlockSpec((1, 1, M), lambda z: (z, 0, 0)),
            pl.BlockSpec((128, 8), lambda z: (0, 0)),
        ],
        out_shape=[
            jax.ShapeDtypeStruct((Zo, Co, M), F32),
            jax.ShapeDtypeStruct((Zo, 1, M), F32),
            jax.ShapeDtypeStruct((128, 8), F32),
        ],
    )(*args)


def _final_apply(y, nm, sc, sh):
    def body(yr, mr, scr, shr, aref):
        a = jnp.maximum(yr[...] * scr[...] + shr[...], 0.0)
        aref[...] = a * mr[...]

    return pl.pallas_call(
        body,
        out_shape=jax.ShapeDtypeStruct(y.shape, F32),
    )(y, nm, sc, sh)


def _finalize(part, g, b):
    Co = g.shape[0]
    cnt = jnp.maximum(part[0, 2], 1.0)
    mean = part[:Co, 0] / cnt
    var = jnp.maximum(part[:Co, 1] / cnt - mean * mean, 0.0)
    scale = g * jax.lax.rsqrt(var + EPS)
    shift = b - mean * scale
    return scale[:, None], shift[:, None]


def _compact(yf, nmf, Whi, Wlo):
    Zo, Co, _ = yf.shape
    Ghi, Phi = Whi + 1, Whi * Whi
    Glo = Wlo + 1
    core = yf[:, :, Ghi:Ghi + Phi].reshape(Zo, Co, Whi, Whi)
    sub = core[:, :, 1:Whi - 1:2, 1:Whi - 1:2]
    y = jnp.pad(sub, ((0, 0), (0, 0), (1, 1), (1, 1)))
    y = y.reshape(Zo, Co, Wlo * Wlo)
    y = jnp.pad(y, ((0, 0), (0, 0), (Glo, Glo)))
    mcore = nmf[:, 0, Ghi:Ghi + Phi].reshape(Zo, Whi, Whi)
    msub = mcore[:, 1:Whi - 1:2, 1:Whi - 1:2]
    nm = jnp.pad(msub, ((0, 0), (1, 1), (1, 1))).reshape(Zo, 1, Wlo * Wlo)
    nm = jnp.pad(nm, ((0, 0), (0, 0), (Glo, Glo)))
    return y, nm


def _selmask(Wf, stride2):
    G = Wf + 1
    i = jnp.arange(Wf)
    if stride2:
        oy = ((i % 2 == 1) & (i < Wf - 1)).astype(F32)
    else:
        oy = ((i > 0) & (i < Wf - 1)).astype(F32)
    sel = oy[:, None] * oy[None, :]
    return jnp.pad(sel.reshape(-1), (G, G))[None]


def kernel(voxel_features, voxel_coords, batch_size, params):
    p = params
    feats = voxel_features.astype(F32)
    zv = voxel_coords[:, 1]
    fv = 83 + (voxel_coords[:, 2] + 1) * 82 + (voxel_coords[:, 3] + 1)
    M1 = 82 * 82 + 2 * 83
    dense = jnp.zeros((ZD, 4, M1), F32).at[zv, :, fv].set(feats)
    mask = jnp.zeros((ZD, 1, M1), F32).at[zv, 0, fv].set(1.0)

    wt = lambda w: jnp.transpose(w, (4, 0, 1, 2, 3)).reshape(w.shape[4], -1)
    int82, int42, int12 = _selmask(82, False), _selmask(42, False), \
        _selmask(12, False)
    odd82, odd42, odd22 = _selmask(82, True), _selmask(42, True), \
        _selmask(22, True)

    y, nm, part = _conv_layer(dense, mask, wt(p['win']), Zi=ZD, Zo=ZD, s=1,
                              zoff=-1, Wf=82, kyx=3, act=False, subm=True)
    sc, sh = _finalize(part, p['gin'], p['bin'])
    y, nm, part = _conv_layer(y, nm, wt(p['w1']), Zi=ZD, Zo=ZD, s=1,
                              zoff=-1, Wf=82, kyx=3, act=True, subm=False,
                              sc=sc, sh=sh, sel=int82)
    sc, sh = _finalize(part, p['g1'], p['b1'])
    y, nm, part = _conv_layer(y, nm, wt(p['w2a']), Zi=ZD, Zo=21, s=2,
                              zoff=-1, Wf=82, kyx=3, act=True, subm=False,
                              sc=sc, sh=sh, sel=odd82)
    y, nm = _compact(y, nm, 82, 42)
    sc, sh = _finalize(part, p['g2a'], p['b2a'])
    y, nm, part = _conv_layer(y, nm, wt(p['w2b']), Zi=21, Zo=21, s=1,
                              zoff=-1, Wf=42, kyx=3, act=True, subm=True,
                              sc=sc, sh=sh)
    sc, sh = _finalize(part, p['g2b'], p['b2b'])
    y, nm, part = _conv_layer(y, nm, wt(p['w2c']), Zi=21, Zo=21, s=1,
                              zoff=-1, Wf=42, kyx=3, act=True, subm=False,
                              sc=sc, sh=sh, sel=int42)
    sc, sh = _finalize(part, p['g2c'], p['b2c'])
    y, nm, part = _conv_layer(y, nm, wt(p['w3a']), Zi=21, Zo=11, s=2,
                              zoff=-1, Wf=42, kyx=3, act=True, subm=False,
                              sc=sc, sh=sh, sel=odd42)
    y, nm = _compact(y, nm, 42, 22)
    sc, sh = _finalize(part, p['g3a'], p['b3a'])
    y, nm, part = _conv_layer(y, nm, wt(p['w3b']), Zi=11, Zo=11, s=1,
                              zoff=-1, Wf=22, kyx=3, act=True, subm=True,
                              sc=sc, sh=sh)
    sc, sh = _finalize(part, p['g3b'], p['b3b'])
    y, nm, part = _conv_layer(y, nm, wt(p['w3c']), Zi=11, Zo=11, s=1,
                              zoff=-1, Wf=22, kyx=3, act=True, subm=True,
                              sc=sc, sh=sh)
    sc, sh = _finalize(part, p['g3c'], p['b3c'])
    y, nm, part = _conv_layer(y, nm, wt(p['w4a']), Zi=11, Zo=5, s=2,
                              zoff=0, Wf=22, kyx=3, act=True, subm=False,
                              sc=sc, sh=sh, sel=odd22)
    y, nm = _compact(y, nm, 22, 12)
    sc, sh = _finalize(part, p['g4a'], p['b4a'])
    y, nm, part = _conv_layer(y, nm, wt(p['w4b']), Zi=5, Zo=5, s=1,
                              zoff=-1, Wf=12, kyx=3, act=True, subm=True,
                              sc=sc, sh=sh)
    sc, sh = _finalize(part, p['g4b'], p['b4b'])
    y, nm, part = _conv_layer(y, nm, wt(p['w4c']), Zi=5, Zo=5, s=1,
                              zoff=-1, Wf=12, kyx=3, act=True, subm=True,
                              sc=sc, sh=sh)
    sc, sh = _finalize(part, p['g4c'], p['b4c'])
    y, nm, part = _conv_layer(y, nm, wt(p['wout']), Zi=5, Zo=2, s=2,
                              zoff=0, Wf=12, kyx=1, act=True, subm=False,
                              sc=sc, sh=sh, sel=int12)
    sc, sh = _finalize(part, p['gout'], p['bout'])
    a = _final_apply(y, nm, sc, sh)

    core = a[:, :, 13:13 + 144].reshape(2, 128, 12, 12)[:, :, 1:11, 1:11]
    return core.transpose(0, 2, 3, 1)[None]
